# Initial kernel scaffold; baseline (speedup 1.0000x reference)
import jax
import jax.numpy as jnp
from jax.experimental import pallas as pl


def kernel(x, edge_attr, i, pool_indices, pool_edges, unpooled_edge_index, n_nodes):
    raise NotImplementedError("write your pallas kernel here")


# speedup vs baseline: 1.0620x; 1.0620x over previous
import functools

import jax
import jax.numpy as jnp
from jax import lax
from jax.experimental import pallas as pl
from jax.experimental.pallas import tpu as pltpu
from jax.experimental.pallas import tpu_sc as plsc

NC = 2
NS = 16
NW = NC * NS
L = 16

_SENT = jnp.iinfo(jnp.int32).max


def _unpool_one(src, idx, n_out, c_i, b_rows):
    m, d = src.shape
    r = n_out // NW
    assert n_out % NW == 0
    nblk = (r + b_rows - 1) // b_rows
    tail = r - (nblk - 1) * b_rows
    r16 = nblk * b_rows
    steps = m.bit_length()

    m_pad = ((m + c_i + 16 + 15) // 16) * 16
    idx_pad = jnp.concatenate([idx, jnp.full((m_pad - m,), _SENT, jnp.int32)])
    src_pad = jnp.concatenate([src, jnp.zeros((8, d), src.dtype)])

    mesh = plsc.VectorSubcoreMesh(core_axis_name="c", subcore_axis_name="s")

    @functools.partial(
        pl.kernel,
        out_type=jax.ShapeDtypeStruct((n_out, d), jnp.float32),
        mesh=mesh,
        scratch_types=[
            pltpu.VMEM((r16,), jnp.int32),
            pltpu.VMEM((c_i + 16,), jnp.int32),
            pltpu.VMEM((16,), jnp.int32),
            pltpu.VMEM((16,), jnp.int32),
            pltpu.VMEM((b_rows, d), jnp.float32),
            pltpu.SemaphoreType.DMA,
            pltpu.SemaphoreType.DMA,
        ],
        compiler_params=pltpu.CompilerParams(needs_layout_passes=False,
                                             use_tc_tiling_on_sc=False),
    )
    def k(src_hbm, idx_hbm, out_hbm,
          srcmap, idxbuf, sbuf, qbuf, rowbuf, sem, sem2):
        wid = lax.axis_index("s") * NC + lax.axis_index("c")
        r0 = wid * r
        lanes = lax.iota(jnp.int32, 16)

        target = jnp.where(lanes == 0, r0, r0 + r)
        lo = jnp.zeros((16,), jnp.int32)
        hi = jnp.full((16,), m, jnp.int32)
        for _ in range(steps):
            mid = lax.div(lo + hi, 2)
            qbuf[...] = mid
            pltpu.async_copy(idx_hbm.at[qbuf], sbuf, sem).wait()
            val = sbuf[...]
            go = lo < hi
            lo = jnp.where(go & (val < target), mid + 1, lo)
            hi = jnp.where(go & (val >= target), mid, hi)
        s = lo[0]
        e = lo[1]

        def init_body(ii, _):
            srcmap[pl.ds(ii * 16, 16)] = jnp.full((16,), m, jnp.int32)
            return 0
        lax.fori_loop(0, r16 // 16, init_body, 0)

        a0 = lax.div(s, 8) * 8
        nchunks = jnp.maximum(lax.div(e - a0 + (c_i - 1), c_i), 0)

        def chunk_body(ci, _):
            base = a0 + ci * c_i
            pltpu.sync_copy(idx_hbm.at[pl.ds(base, c_i + 16)], idxbuf)

            def vec_body(j, _):
                cur = idxbuf[pl.ds(j * 16, 16)]
                nxt = idxbuf[pl.ds(j * 16 + 1, 16)]
                kpos = base + j * 16 + lanes
                valid = (kpos >= s) & (kpos < e) & (cur != nxt)
                local = jnp.where(valid, cur - r0, 0)
                plsc.store_scatter(srcmap, [local], kpos, mask=valid)
                return 0
            lax.fori_loop(0, c_i // 16, vec_body, 0)
            return 0
        lax.fori_loop(0, nchunks, chunk_body, 0)

        def blk_body(bi, _):
            off = bi * b_rows
            pltpu.async_copy(
                src_hbm.at[srcmap.at[pl.ds(off, b_rows)]], rowbuf, sem2).wait()
            pltpu.sync_copy(rowbuf, out_hbm.at[pl.ds(r0 + off, b_rows), :])
            return 0
        lax.fori_loop(0, nblk - 1, blk_body, 0)
        off = (nblk - 1) * b_rows
        pltpu.async_copy(
            src_hbm.at[srcmap.at[pl.ds(off, b_rows)]], rowbuf, sem2).wait()
        pltpu.sync_copy(rowbuf.at[pl.ds(0, tail), :],
                        out_hbm.at[pl.ds(r0 + off, tail), :])

    return k(src_pad, idx_pad)


def kernel(x, edge_attr, i, pool_indices, pool_edges, unpooled_edge_index,
           n_nodes):
    new_x = _unpool_one(x, pool_indices, 100000, c_i=2048, b_rows=128)
    new_edge_attr = _unpool_one(edge_attr, pool_edges,
                                unpooled_edge_index.shape[1],
                                c_i=4096, b_rows=128)
    return new_x, new_edge_attr


# speedup vs baseline: 1.0622x; 1.0002x over previous
import functools

import jax
import jax.numpy as jnp
from jax import lax
from jax.experimental import pallas as pl
from jax.experimental.pallas import tpu as pltpu
from jax.experimental.pallas import tpu_sc as plsc

NC = 2
NS = 16
NW = NC * NS
L = 16

_SENT = jnp.iinfo(jnp.int32).max


def _unpool_one(src, idx, n_out, c_i, b_rows, nb):
    m, d = src.shape
    r = n_out // NW
    assert n_out % NW == 0
    nblk = (r + b_rows - 1) // b_rows
    tail = r - (nblk - 1) * b_rows
    r16 = nblk * b_rows
    steps = m.bit_length()
    full = nblk - 1
    assert full % nb == 0
    groups = full // nb

    m_pad = ((m + c_i + 16 + 15) // 16) * 16
    idx_pad = jnp.concatenate([idx, jnp.full((m_pad - m,), _SENT, jnp.int32)])
    src_pad = jnp.concatenate([src, jnp.zeros((8, d), src.dtype)])

    mesh = plsc.VectorSubcoreMesh(core_axis_name="c", subcore_axis_name="s")

    @functools.partial(
        pl.kernel,
        out_type=jax.ShapeDtypeStruct((n_out, d), jnp.float32),
        mesh=mesh,
        scratch_types=[
            pltpu.VMEM((r16,), jnp.int32),
            pltpu.VMEM((c_i + 16,), jnp.int32),
            pltpu.VMEM((16,), jnp.int32),
            pltpu.VMEM((16,), jnp.int32),
        ] + [pltpu.VMEM((b_rows, d), jnp.float32) for _ in range(nb)]
          + [pltpu.SemaphoreType.DMA for _ in range(2 * nb + 1)],
        compiler_params=pltpu.CompilerParams(needs_layout_passes=False,
                                             use_tc_tiling_on_sc=False),
    )
    def k(src_hbm, idx_hbm, out_hbm, srcmap, idxbuf, sbuf, qbuf, *bufsem):
        rowbufs = bufsem[:nb]
        gsems = bufsem[nb:2 * nb]
        wsems = bufsem[2 * nb:3 * nb]
        sem = bufsem[3 * nb]
        wid = lax.axis_index("s") * NC + lax.axis_index("c")
        r0 = wid * r
        lanes = lax.iota(jnp.int32, 16)

        target = jnp.where(lanes == 0, r0, r0 + r)
        lo = jnp.zeros((16,), jnp.int32)
        hi = jnp.full((16,), m, jnp.int32)
        for _ in range(steps):
            mid = lax.div(lo + hi, 2)
            qbuf[...] = mid
            pltpu.async_copy(idx_hbm.at[qbuf], sbuf, sem).wait()
            val = sbuf[...]
            go = lo < hi
            lo = jnp.where(go & (val < target), mid + 1, lo)
            hi = jnp.where(go & (val >= target), mid, hi)
        s = lo[0]
        e = lo[1]

        def init_body(ii, _):
            srcmap[pl.ds(ii * 16, 16)] = jnp.full((16,), m, jnp.int32)
            return 0
        lax.fori_loop(0, r16 // 16, init_body, 0)

        a0 = lax.div(s, 8) * 8
        nchunks = jnp.maximum(lax.div(e - a0 + (c_i - 1), c_i), 0)

        def chunk_body(ci, _):
            base = a0 + ci * c_i
            pltpu.sync_copy(idx_hbm.at[pl.ds(base, c_i + 16)], idxbuf)

            def vec_body(j, _):
                cur = idxbuf[pl.ds(j * 16, 16)]
                nxt = idxbuf[pl.ds(j * 16 + 1, 16)]
                kpos = base + j * 16 + lanes
                valid = (kpos >= s) & (kpos < e) & (cur != nxt)
                local = jnp.where(valid, cur - r0, 0)
                plsc.store_scatter(srcmap, [local], kpos, mask=valid)
                return 0
            lax.fori_loop(0, c_i // 16, vec_body, 0)
            return 0
        lax.fori_loop(0, nchunks, chunk_body, 0)

        def g_start(off, b):
            pltpu.async_copy(
                src_hbm.at[srcmap.at[pl.ds(off, b_rows)]], rowbufs[b], gsems[b])

        def g_wait(off, b):
            pltpu.make_async_copy(
                src_hbm.at[srcmap.at[pl.ds(off, b_rows)]], rowbufs[b],
                gsems[b]).wait()

        def w_start(off, b):
            pltpu.async_copy(
                rowbufs[b], out_hbm.at[pl.ds(r0 + off, b_rows), :], wsems[b])

        def w_wait(off, b):
            pltpu.make_async_copy(
                rowbufs[b], out_hbm.at[pl.ds(r0 + off, b_rows), :],
                wsems[b]).wait()

        for b in range(nb):
            g_start(b * b_rows, b)

        def group(i, _):
            base = i * nb * b_rows
            for b in range(nb):
                off = base + b * b_rows
                g_wait(off, b)
                w_start(off, b)

            @pl.when(i < groups - 1)
            def _():
                for b in range(nb):
                    off = base + b * b_rows
                    w_wait(off, b)
                    g_start(off + nb * b_rows, b)
            return 0
        lax.fori_loop(0, groups, group, 0)
        last_base = (groups - 1) * nb * b_rows
        for b in range(nb):
            w_wait(last_base + b * b_rows, b)

        off = full * b_rows
        pltpu.async_copy(
            src_hbm.at[srcmap.at[pl.ds(off, b_rows)]], rowbufs[0],
            gsems[0]).wait()
        pltpu.sync_copy(rowbufs[0].at[pl.ds(0, tail), :],
                        out_hbm.at[pl.ds(r0 + off, tail), :])

    return k(src_pad, idx_pad)


def kernel(x, edge_attr, i, pool_indices, pool_edges, unpooled_edge_index,
           n_nodes):
    new_x = _unpool_one(x, pool_indices, 100000, c_i=2048, b_rows=128, nb=4)
    new_edge_attr = _unpool_one(edge_attr, pool_edges,
                                unpooled_edge_index.shape[1],
                                c_i=4096, b_rows=128, nb=6)
    return new_x, new_edge_attr


# speedup vs baseline: 4.6648x; 4.3917x over previous
import functools

import jax
import jax.numpy as jnp
from jax import lax
from jax.experimental import pallas as pl
from jax.experimental.pallas import tpu as pltpu
from jax.experimental.pallas import tpu_sc as plsc

NC = 2
NS = 16
NW = NC * NS
L = 16

_SENT = jnp.iinfo(jnp.int32).max


def _unpool_one(src, idx, n_out, c_i, b_rows, nb, npad):
    m, d = src.shape
    r = n_out // NW
    assert n_out % NW == 0
    nblk = (r + b_rows - 1) // b_rows
    tail = r - (nblk - 1) * b_rows
    r16 = nblk * b_rows
    steps = m.bit_length()
    full = nblk - 1
    assert full % nb == 0
    groups = full // nb

    m_pad = ((m + c_i + 16 + 15) // 16) * 16
    idx_pad = jnp.concatenate([idx, jnp.full((m_pad - m,), _SENT, jnp.int32)])
    src_pad = jnp.concatenate([src, jnp.zeros((npad, d), src.dtype)])

    mesh = plsc.VectorSubcoreMesh(core_axis_name="c", subcore_axis_name="s")

    @functools.partial(
        pl.kernel,
        out_type=jax.ShapeDtypeStruct((n_out, d), jnp.float32),
        mesh=mesh,
        scratch_types=[
            pltpu.VMEM((r16,), jnp.int32),
            pltpu.VMEM((c_i + 16,), jnp.int32),
            pltpu.VMEM((16,), jnp.int32),
            pltpu.VMEM((16,), jnp.int32),
        ] + [pltpu.VMEM((b_rows, d), jnp.float32) for _ in range(nb)]
          + [pltpu.SemaphoreType.DMA for _ in range(2 * nb + 1)],
        compiler_params=pltpu.CompilerParams(needs_layout_passes=False,
                                             use_tc_tiling_on_sc=False),
    )
    def k(src_hbm, idx_hbm, out_hbm, srcmap, idxbuf, sbuf, qbuf, *bufsem):
        rowbufs = bufsem[:nb]
        gsems = bufsem[nb:2 * nb]
        wsems = bufsem[2 * nb:3 * nb]
        sem = bufsem[3 * nb]
        wid = lax.axis_index("s") * NC + lax.axis_index("c")
        r0 = wid * r
        lanes = lax.iota(jnp.int32, 16)

        target = jnp.where(lanes == 0, r0, r0 + r)
        lo = jnp.zeros((16,), jnp.int32)
        hi = jnp.full((16,), m, jnp.int32)
        for _ in range(steps):
            mid = lax.div(lo + hi, 2)
            qbuf[...] = mid
            pltpu.async_copy(idx_hbm.at[qbuf], sbuf, sem).wait()
            val = sbuf[...]
            go = lo < hi
            lo = jnp.where(go & (val < target), mid + 1, lo)
            hi = jnp.where(go & (val >= target), mid, hi)
        s = lo[0]
        e = lo[1]

        sent = m + wid * (npad // NW)
        def init_body(ii, _):
            srcmap[pl.ds(ii * 16, 16)] = jnp.full((16,), 0, jnp.int32) + sent
            return 0
        lax.fori_loop(0, r16 // 16, init_body, 0)

        a0 = lax.div(s, 8) * 8
        nchunks = jnp.maximum(lax.div(e - a0 + (c_i - 1), c_i), 0)

        def chunk_body(ci, _):
            base = a0 + ci * c_i
            pltpu.sync_copy(idx_hbm.at[pl.ds(base, c_i + 16)], idxbuf)

            def vec_body(j, _):
                cur = idxbuf[pl.ds(j * 16, 16)]
                nxt = idxbuf[pl.ds(j * 16 + 1, 16)]
                kpos = base + j * 16 + lanes
                valid = (kpos >= s) & (kpos < e) & (cur != nxt)
                local = jnp.where(valid, cur - r0, 0)
                plsc.store_scatter(srcmap, [local], kpos, mask=valid)
                return 0
            lax.fori_loop(0, c_i // 16, vec_body, 0)
            return 0
        lax.fori_loop(0, nchunks, chunk_body, 0)

        def g_start(off, b):
            pltpu.async_copy(
                src_hbm.at[srcmap.at[pl.ds(off, b_rows)]], rowbufs[b], gsems[b])

        def g_wait(off, b):
            pltpu.make_async_copy(
                src_hbm.at[srcmap.at[pl.ds(off, b_rows)]], rowbufs[b],
                gsems[b]).wait()

        def w_start(off, b):
            pltpu.async_copy(
                rowbufs[b], out_hbm.at[pl.ds(r0 + off, b_rows), :], wsems[b])

        def w_wait(off, b):
            pltpu.make_async_copy(
                rowbufs[b], out_hbm.at[pl.ds(r0 + off, b_rows), :],
                wsems[b]).wait()

        for b in range(nb):
            g_start(b * b_rows, b)

        def group(i, _):
            base = i * nb * b_rows
            for b in range(nb):
                off = base + b * b_rows
                g_wait(off, b)
                w_start(off, b)

            @pl.when(i < groups - 1)
            def _():
                for b in range(nb):
                    off = base + b * b_rows
                    w_wait(off, b)
                    g_start(off + nb * b_rows, b)
            return 0
        lax.fori_loop(0, groups, group, 0)
        last_base = (groups - 1) * nb * b_rows
        for b in range(nb):
            w_wait(last_base + b * b_rows, b)

        off = full * b_rows
        pltpu.async_copy(
            src_hbm.at[srcmap.at[pl.ds(off, b_rows)]], rowbufs[0],
            gsems[0]).wait()
        pltpu.sync_copy(rowbufs[0].at[pl.ds(0, tail), :],
                        out_hbm.at[pl.ds(r0 + off, tail), :])

    return k(src_pad, idx_pad)


def kernel(x, edge_attr, i, pool_indices, pool_edges, unpooled_edge_index,
           n_nodes):
    new_x = _unpool_one(x, pool_indices, 100000, c_i=2048, b_rows=128, nb=4,
                        npad=1024)
    new_edge_attr = _unpool_one(edge_attr, pool_edges,
                                unpooled_edge_index.shape[1],
                                c_i=4096, b_rows=128, nb=6, npad=4096)
    return new_x, new_edge_attr
